# baseline (speedup 1.0000x reference)
import functools

import jax
import jax.numpy as jnp
from jax import lax
from jax.experimental import pallas as pl
from jax.experimental.pallas import tpu as pltpu
from jax.experimental.pallas import tpu_sc as plsc

NUM_SC_CORES = 2
NUM_SUBCORES = 16
NUM_WORKERS = NUM_SC_CORES * NUM_SUBCORES
LANES = 16
WIDE = 16
NSEG = 16
LN2 = 0.6931471805599453


def _dense_body(x_ref, w_ref, b_ref, out_ref):
    y = jnp.dot(x_ref[...], w_ref[...], preferred_element_type=jnp.float32)
    y = y + b_ref[...]
    y8 = y[:, 0:8]
    amax = jnp.max(y8, axis=-1, keepdims=True)
    lse = jnp.log(jnp.sum(jnp.exp(y8 - amax), axis=-1, keepdims=True)) + amax
    lanei = lax.broadcasted_iota(jnp.int32, (1, WIDE), 1)
    out_ref[...] = jnp.where(lanei < 8, y - lse, y)


def _dense_call(x, w_cat, b_cat, block_rows):
    n, e = x.shape
    return pl.pallas_call(
        _dense_body,
        grid=(n // block_rows,),
        in_specs=[
            pl.BlockSpec((block_rows, e), lambda i: (i, 0)),
            pl.BlockSpec((e, WIDE), lambda i: (0, 0)),
            pl.BlockSpec((1, WIDE), lambda i: (0, 0)),
        ],
        out_specs=pl.BlockSpec((block_rows, WIDE), lambda i: (i, 0)),
        out_shape=jax.ShapeDtypeStruct((n, WIDE), jnp.float32),
    )(x, w_cat, b_cat)


def _part_body(chunk, main_hbm, idx_hbm, pmax_hbm, psum_hbm,
               mv, iv, dv, lm_ref, s_ref):
    minf = jnp.float32(-jnp.inf)
    wid = lax.axis_index("s") * NUM_SC_CORES + lax.axis_index("c")
    base = wid * chunk
    pltpu.sync_copy(main_hbm.at[pl.ds(base * WIDE, chunk * WIDE)], mv)
    pltpu.sync_copy(idx_hbm.at[pl.ds(base, chunk)], iv)

    nvec = chunk // LANES
    lane = lax.iota(jnp.int32, LANES)
    lane_w = lane * WIDE

    def compact(j, c):
        dv[pl.ds(j * LANES, LANES)] = plsc.load_gather(
            mv, [j * (LANES * WIDE) + lane_w + 8])
        return c

    lax.fori_loop(0, nvec, compact, 0)

    s_ref[...] = jnp.zeros((LANES,), jnp.float32)

    first = iv[pl.ds(0, LANES)][0]
    last = iv[pl.ds(chunk - LANES, LANES)][LANES - 1]

    @pl.when(first == last)
    def _chunk_uniform():
        def body(i, acc):
            return jnp.maximum(acc, dv[pl.ds(i * LANES, LANES)])

        acc = lax.fori_loop(0, nvec, body, jnp.full((LANES,), minf, jnp.float32))
        lm_ref[...] = jnp.where(lane == first, jnp.max(acc), minf)

    @pl.when(first != last)
    def _chunk_mixed():
        def body(i, lm):
            v = dv[pl.ds(i * LANES, LANES)]
            sg = iv[pl.ds(i * LANES, LANES)]
            s0 = sg[0]
            s15 = sg[LANES - 1]

            def vec_uniform(lm):
                return jnp.where(lane == s0, jnp.maximum(lm, jnp.max(v)), lm)

            def vec_mixed(lm):
                def seg_loop(b, lm):
                    mb = jnp.max(jnp.where(sg == b, v, minf))
                    return jnp.where(lane == b, jnp.maximum(lm, mb), lm)

                return lax.fori_loop(0, NSEG, seg_loop, lm)

            return lax.cond(s0 == s15, vec_uniform, vec_mixed, lm)

        lm_ref[...] = lax.fori_loop(
            0, nvec, body, jnp.full((LANES,), minf, jnp.float32))

    def body2(i, c):
        v = dv[pl.ds(i * LANES, LANES)]
        sg = iv[pl.ds(i * LANES, LANES)]
        shift = plsc.load_gather(lm_ref, [sg])
        plsc.addupdate_scatter(s_ref, [sg], jnp.exp(v - shift))
        return c

    lax.fori_loop(0, nvec, body2, 0)

    pltpu.sync_copy(lm_ref, pmax_hbm.at[pl.ds(wid * LANES, LANES)])
    pltpu.sync_copy(s_ref, psum_hbm.at[pl.ds(wid * LANES, LANES)])


def _part_call(main_flat, idx):
    n = idx.shape[0]
    chunk = n // NUM_WORKERS
    mesh = plsc.VectorSubcoreMesh(
        core_axis_name="c", subcore_axis_name="s",
        num_cores=NUM_SC_CORES, num_subcores=NUM_SUBCORES,
    )
    return pl.kernel(
        functools.partial(_part_body, chunk),
        out_type=[
            jax.ShapeDtypeStruct((NUM_WORKERS * LANES,), jnp.float32),
            jax.ShapeDtypeStruct((NUM_WORKERS * LANES,), jnp.float32),
        ],
        mesh=mesh,
        compiler_params=pltpu.CompilerParams(needs_layout_passes=False),
        scratch_types=[
            pltpu.VMEM((chunk * WIDE,), jnp.float32),
            pltpu.VMEM((chunk,), jnp.int32),
            pltpu.VMEM((chunk,), jnp.float32),
            pltpu.VMEM((LANES,), jnp.float32),
            pltpu.VMEM((LANES,), jnp.float32),
        ],
    )(main_flat, idx)


def _comb_body(chunk, main_hbm, idx_hbm, pmax_hbm, psum_hbm, out_hbm,
               mv, iv, pmv, psv, clse_ref, c_ref, ob):
    wid = lax.axis_index("s") * NUM_SC_CORES + lax.axis_index("c")
    base = wid * chunk
    pltpu.sync_copy(main_hbm.at[pl.ds(base * WIDE, chunk * WIDE)], mv)
    pltpu.sync_copy(idx_hbm.at[pl.ds(base, chunk)], iv)
    pltpu.sync_copy(pmax_hbm, pmv)
    pltpu.sync_copy(psum_hbm, psv)

    minf = jnp.float32(-jnp.inf)
    lane = lax.iota(jnp.int32, LANES)
    lane_w = lane * WIDE

    def mbody(w, m):
        return jnp.maximum(m, pmv[pl.ds(w * LANES, LANES)])

    m = lax.fori_loop(0, NUM_WORKERS, mbody, jnp.full((LANES,), minf, jnp.float32))

    def sbody(w, s):
        return s + psv[pl.ds(w * LANES, LANES)] * jnp.exp(
            pmv[pl.ds(w * LANES, LANES)] - m)

    s = lax.fori_loop(0, NUM_WORKERS, sbody, jnp.zeros((LANES,), jnp.float32))

    bits = plsc.bitcast(s, jnp.int32)
    y = ((bits >> 23) - 127).astype(jnp.float32) * jnp.float32(LN2)
    for _ in range(4):
        y = y + s * jnp.exp(-y) - 1.0
    clse_ref[...] = m + y

    nvec = chunk // LANES

    def cbody(j, c):
        sg = iv[pl.ds(j * LANES, LANES)]
        d = plsc.load_gather(mv, [j * (LANES * WIDE) + lane_w + 8])
        cl = plsc.load_gather(clse_ref, [sg])
        c_ref[pl.ds(j * LANES, LANES)] = d - cl
        return c

    lax.fori_loop(0, nvec, cbody, 0)

    def obody(j, c):
        o = j * LANES + lane
        la = plsc.load_gather(mv, [o * 2 - (o & 7)])
        cr = plsc.load_gather(c_ref, [o >> 3])
        ob[pl.ds(j * LANES, LANES)] = la + cr
        return c

    lax.fori_loop(0, nvec * 8, obody, 0)
    pltpu.sync_copy(ob, out_hbm.at[pl.ds(base * 8, chunk * 8)])


def _comb_call(main_flat, idx, pmax, psum):
    n = idx.shape[0]
    chunk = n // NUM_WORKERS
    mesh = plsc.VectorSubcoreMesh(
        core_axis_name="c", subcore_axis_name="s",
        num_cores=NUM_SC_CORES, num_subcores=NUM_SUBCORES,
    )
    return pl.kernel(
        functools.partial(_comb_body, chunk),
        out_type=jax.ShapeDtypeStruct((n * 8,), jnp.float32),
        mesh=mesh,
        compiler_params=pltpu.CompilerParams(needs_layout_passes=False),
        scratch_types=[
            pltpu.VMEM((chunk * WIDE,), jnp.float32),
            pltpu.VMEM((chunk,), jnp.int32),
            pltpu.VMEM((NUM_WORKERS * LANES,), jnp.float32),
            pltpu.VMEM((NUM_WORKERS * LANES,), jnp.float32),
            pltpu.VMEM((LANES,), jnp.float32),
            pltpu.VMEM((chunk,), jnp.float32),
            pltpu.VMEM((chunk * 8,), jnp.float32),
        ],
    )(main_flat, idx, pmax, psum)


def kernel(embed_states, batch_index, W_dev, b_dev, W_act, b_act):
    n, e = embed_states.shape
    a = W_act.shape[1]
    idx = batch_index.astype(jnp.int32)
    w_cat = jnp.zeros((e, WIDE), jnp.float32)
    w_cat = w_cat.at[:, :a].set(W_act).at[:, a].set(W_dev[:, 0])
    b_cat = jnp.zeros((1, WIDE), jnp.float32)
    b_cat = b_cat.at[0, :a].set(b_act).at[0, a].set(b_dev[0])
    main = _dense_call(embed_states, w_cat, b_cat, block_rows=2048)
    main_flat = main.reshape(-1)
    pmax, psum = _part_call(main_flat, idx)
    out_flat = _comb_call(main_flat, idx, pmax, psum)
    return out_flat.reshape(n, a)
